# Initial kernel scaffold; baseline (speedup 1.0000x reference)
import jax
import jax.numpy as jnp
from jax.experimental import pallas as pl


def kernel(x, W_embed, b_embed, g_norm, W_q, W_k, W_v, k_pos, v_pos, Wc_k, bc_k, Wc_v, bc_v, null_k, null_v, W_gate, b_gate, W_o, b_o, W_h1, b_h1, W_h2, b_h2):
    raise NotImplementedError("write your pallas kernel here")


# speedup vs baseline: 31.2042x; 31.2042x over previous
import functools

import jax
import jax.numpy as jnp
from jax import lax
from jax.experimental import pallas as pl

_B, _N = 8, 2048
_DIM, _DH, _H = 64, 32, 6
_SW, _CBS, _SBS = 7, 16, 8
_NC = _N // _CBS
_OUT = 7
_SCALE = _DH ** -0.5
_NEG = -1e30


def _proj_kernel(x_ref, we_ref, be_ref, gn_ref, wq_ref, wk_ref, wv_ref,
                 wg_ref, bg_ref, qo_ref, ko_ref, vo_ref, go_ref):
    t = x_ref[0] * we_ref[...] + be_ref[...]
    hn = t * lax.rsqrt(jnp.mean(t * t, axis=1, keepdims=True) + 1e-6)
    hn = hn * gn_ref[...]
    qf = jnp.dot(hn, wq_ref[...], preferred_element_type=jnp.float32)
    kf = jnp.dot(hn, wk_ref[...], preferred_element_type=jnp.float32)
    vf = jnp.dot(hn, wv_ref[...], preferred_element_type=jnp.float32)
    for h in range(_H):
        sl = slice(h * _DH, (h + 1) * _DH)
        qo_ref[0, h] = qf[:, sl]
        ko_ref[0, h] = kf[:, sl]
        vo_ref[0, h] = vf[:, sl]
    go_ref[0] = jax.nn.sigmoid(
        jnp.dot(hn, wg_ref[...], preferred_element_type=jnp.float32)
        + bg_ref[...])


def _attn_kernel(q_ref, k_ref, v_ref, k4_ref, v4_ref, kp_ref, vp_ref,
                 wck_ref, bck_ref, wcv_ref, bcv_ref, nk_ref, nv_ref, g_ref,
                 out_ref):
    f32 = jnp.float32
    qf = q_ref[0, 0]
    kf = k_ref[0, 0]
    vf = v_ref[0, 0]
    k4 = k4_ref[0, 0]
    v4 = v4_ref[0, 0]

    ck = jnp.dot(k4 + kp_ref[0], wck_ref[...],
                 preferred_element_type=f32) + bck_ref[...]
    cv = jnp.dot(v4 + vp_ref[0], wcv_ref[...],
                 preferred_element_type=f32) + bcv_ref[...]

    simc = lax.dot_general(qf, ck, (((1,), (1,)), ((), ())),
                           preferred_element_type=f32) * _SCALE
    nl = jnp.sum(qf * nk_ref[0], axis=1, keepdims=True) * _SCALE
    n_row = lax.broadcasted_iota(jnp.int32, (_N, 1), 0)
    j_id = lax.broadcasted_iota(jnp.int32, (_N, _NC), 1)
    nvalid = (n_row + 1) // _CBS
    msim = jnp.where(j_id < nvalid, simc, _NEG)
    mb = jnp.max(msim, axis=1, keepdims=True)
    m = jnp.maximum(mb, nl)
    eb = jnp.exp(msim - m)
    en = jnp.exp(nl - m)
    den = jnp.sum(eb, axis=1, keepdims=True) + en
    outc = (lax.dot_general(eb, cv, (((1,), (0,)), ((), ())),
                            preferred_element_type=f32)
            + en * nv_ref[0]) / den

    jm = jnp.min(jnp.where(msim == mb, j_id, _NC), axis=1, keepdims=True)
    onehot = (j_id == jm).astype(f32)
    gk = jnp.dot(onehot, k4, preferred_element_type=f32)
    gv = jnp.dot(onehot, v4, preferred_element_type=f32)
    sims = []
    for s in range(_CBS):
        sl = slice(s * _DH, (s + 1) * _DH)
        sims.append(jnp.sum(qf * gk[:, sl], axis=1, keepdims=True))
    sim16 = jnp.concatenate(sims, axis=1) * _SCALE
    s_id = lax.broadcasted_iota(jnp.int32, (_N, _CBS), 1)
    sim16 = jnp.where(jm * _CBS + s_id <= n_row, sim16, _NEG)

    shk = [kf]
    shv = [vf]
    for t in range(1, _SBS):
        pad = jnp.zeros((t, _DH), f32)
        shk.append(jnp.concatenate([pad, kf[:_N - t]], axis=0))
        shv.append(jnp.concatenate([pad, vf[:_N - t]], axis=0))
    d8 = jnp.concatenate(
        [jnp.sum(qf * shk[t], axis=1, keepdims=True) for t in range(_SBS)],
        axis=1) * _SCALE
    t8 = lax.broadcasted_iota(jnp.int32, (_N, _SBS), 1)
    own = jnp.where(t8 <= n_row % _SBS, d8, _NEG)

    mf = jnp.maximum(jnp.max(sim16, axis=1, keepdims=True),
                     jnp.max(own, axis=1, keepdims=True))
    e16 = jnp.exp(sim16 - mf)
    e8 = jnp.exp(own - mf)
    denf = jnp.sum(e16, axis=1, keepdims=True) + jnp.sum(e8, axis=1,
                                                         keepdims=True)
    outf = jnp.zeros((_N, _DH), f32)
    for s in range(_CBS):
        outf += e16[:, s:s + 1] * gv[:, s * _DH:(s + 1) * _DH]
    for t in range(_SBS):
        outf += e8[:, t:t + 1] * shv[t]
    outf = outf / denf

    w7 = jnp.where(t8[:, :_SW] <= jnp.minimum(n_row, _SW - 1),
                   d8[:, :_SW], _NEG)
    mw = jnp.max(w7, axis=1, keepdims=True)
    ew = jnp.exp(w7 - mw)
    denw = jnp.sum(ew, axis=1, keepdims=True)
    outw = jnp.zeros((_N, _DH), f32)
    for t in range(_SW):
        outw += ew[:, t:t + 1] * shv[t]
    outw = outw / denw

    g = g_ref[0, 0]
    comb = g[:, 0:1] * outc + g[:, 1:2] * outf + g[:, 2:3] * outw
    out_ref[0, 0] = jnp.sum(comb, axis=0, keepdims=True)


def _head_kernel(acc_ref, wo_ref, bo_ref, w1_ref, b1_ref, w2_ref, b2_ref,
                 o_ref):
    pooled = jnp.dot(acc_ref[...], wo_ref[...],
                     preferred_element_type=jnp.float32) * (1.0 / _N)
    pooled = pooled + bo_ref[...]
    h1 = jax.nn.gelu(jnp.dot(pooled, w1_ref[...],
                             preferred_element_type=jnp.float32) + b1_ref[...])
    o_ref[...] = jnp.dot(h1, w2_ref[...],
                         preferred_element_type=jnp.float32) + b2_ref[...]


@functools.partial(jax.jit, static_argnames=())
def kernel(x, W_embed, b_embed, g_norm, W_q, W_k, W_v, k_pos, v_pos, Wc_k,
           bc_k, Wc_v, bc_v, null_k, null_v, W_gate, b_gate, W_o, b_o, W_h1,
           b_h1, W_h2, b_h2):
    f32 = jnp.float32
    B, N = x.shape
    full = lambda shp: pl.BlockSpec(shp, lambda b: tuple(0 for _ in shp))

    q, k, v, gates = pl.pallas_call(
        _proj_kernel,
        grid=(B,),
        in_specs=[
            pl.BlockSpec((1, N, 1), lambda b: (b, 0, 0)),
            full((1, _DIM)), full((1, _DIM)), full((1, _DIM)),
            full((_DIM, _H * _DH)), full((_DIM, _H * _DH)),
            full((_DIM, _H * _DH)),
            full((_DIM, 3 * _H)), full((1, 3 * _H)),
        ],
        out_specs=[
            pl.BlockSpec((1, _H, N, _DH), lambda b: (b, 0, 0, 0)),
            pl.BlockSpec((1, _H, N, _DH), lambda b: (b, 0, 0, 0)),
            pl.BlockSpec((1, _H, N, _DH), lambda b: (b, 0, 0, 0)),
            pl.BlockSpec((1, N, 3 * _H), lambda b: (b, 0, 0)),
        ],
        out_shape=[
            jax.ShapeDtypeStruct((B, _H, N, _DH), f32),
            jax.ShapeDtypeStruct((B, _H, N, _DH), f32),
            jax.ShapeDtypeStruct((B, _H, N, _DH), f32),
            jax.ShapeDtypeStruct((B, N, 3 * _H), f32),
        ],
    )(x.reshape(B, N, 1), W_embed, b_embed.reshape(1, _DIM),
      g_norm.reshape(1, _DIM), W_q, W_k, W_v, W_gate, b_gate.reshape(1, -1))

    k4 = k.reshape(B, _H, _NC, _CBS * _DH)
    v4 = v.reshape(B, _H, _NC, _CBS * _DH)
    gates_r = gates.reshape(B, N, 3, _H).transpose(0, 3, 1, 2)

    bh_spec = lambda shp: pl.BlockSpec(shp, lambda b, h: (b, h) + (0,) * (len(shp) - 2))
    h_spec = lambda shp: pl.BlockSpec(shp, lambda b, h: (h,) + (0,) * (len(shp) - 1))
    w_spec = lambda shp: pl.BlockSpec(shp, lambda b, h: tuple(0 for _ in shp))

    acc = pl.pallas_call(
        _attn_kernel,
        grid=(B, _H),
        in_specs=[
            bh_spec((1, 1, N, _DH)), bh_spec((1, 1, N, _DH)),
            bh_spec((1, 1, N, _DH)),
            bh_spec((1, 1, _NC, _CBS * _DH)), bh_spec((1, 1, _NC, _CBS * _DH)),
            h_spec((1, 1, _CBS * _DH)), h_spec((1, 1, _CBS * _DH)),
            w_spec((_CBS * _DH, _DH)), w_spec((1, _DH)),
            w_spec((_CBS * _DH, _DH)), w_spec((1, _DH)),
            h_spec((1, 1, _DH)), h_spec((1, 1, _DH)),
            bh_spec((1, 1, N, 3)),
        ],
        out_specs=pl.BlockSpec((1, 1, 1, _DH), lambda b, h: (b, h, 0, 0)),
        out_shape=jax.ShapeDtypeStruct((B, _H, 1, _DH), f32),
    )(q, k, v, k4, v4,
      k_pos.reshape(_H, 1, _CBS * _DH), v_pos.reshape(_H, 1, _CBS * _DH),
      Wc_k, bc_k.reshape(1, _DH), Wc_v, bc_v.reshape(1, _DH),
      null_k.reshape(_H, 1, _DH), null_v.reshape(_H, 1, _DH), gates_r)

    out = pl.pallas_call(
        _head_kernel,
        out_shape=jax.ShapeDtypeStruct((B, _OUT), f32),
    )(acc.reshape(B, _H * _DH), W_o, b_o.reshape(1, _DIM),
      W_h1, b_h1.reshape(1, 32), W_h2, b_h2.reshape(1, _OUT))
    return out


# speedup vs baseline: 78.4031x; 2.5126x over previous
import functools

import jax
import jax.numpy as jnp
from jax import lax
from jax.experimental import pallas as pl
from jax.experimental.pallas import tpu as pltpu

_B, _N = 8, 2048
_DIM, _DH, _H = 64, 32, 6
_SW, _CBS, _SBS = 7, 16, 8
_NC = _N // _CBS
_OUT = 7
_SCALE = _DH ** -0.5
_NEG = -1e30


def _proj_kernel(x_ref, we_ref, be_ref, gn_ref, wq_ref, wk_ref, wv_ref,
                 wg_ref, bg_ref, qo_ref, ko_ref, vo_ref, go_ref):
    t = x_ref[0] * we_ref[...] + be_ref[...]
    hn = t * lax.rsqrt(jnp.mean(t * t, axis=1, keepdims=True) + 1e-6)
    hn = hn * gn_ref[...]
    qf = jnp.dot(hn, wq_ref[...], preferred_element_type=jnp.float32)
    kf = jnp.dot(hn, wk_ref[...], preferred_element_type=jnp.float32)
    vf = jnp.dot(hn, wv_ref[...], preferred_element_type=jnp.float32)
    for h in range(_H):
        sl = slice(h * _DH, (h + 1) * _DH)
        qo_ref[0, h] = qf[:, sl]
        ko_ref[0, h] = kf[:, sl]
        vo_ref[0, h] = vf[:, sl]
    go_ref[0] = jax.nn.sigmoid(
        jnp.dot(hn, wg_ref[...], preferred_element_type=jnp.float32)
        + bg_ref[...])


def _attn_kernel(q_ref, k_ref, v_ref, k4_ref, v4_ref, kp_ref, vp_ref,
                 wck_ref, bck_ref, wcv_ref, bcv_ref, nk_ref, nv_ref, g_ref,
                 out_ref):
    f32 = jnp.float32
    qf = q_ref[0, 0]
    kf = k_ref[0, 0]
    vf = v_ref[0, 0]
    k4 = k4_ref[0, 0]
    v4 = v4_ref[0, 0]

    ck = jnp.dot(k4 + kp_ref[0], wck_ref[...],
                 preferred_element_type=f32) + bck_ref[...]
    cv = jnp.dot(v4 + vp_ref[0], wcv_ref[...],
                 preferred_element_type=f32) + bcv_ref[...]

    simc = lax.dot_general(qf, ck, (((1,), (1,)), ((), ())),
                           preferred_element_type=f32) * _SCALE
    nl = jnp.sum(qf * nk_ref[0], axis=1, keepdims=True) * _SCALE
    n_row = lax.broadcasted_iota(jnp.int32, (_N, 1), 0)
    j_id = lax.broadcasted_iota(jnp.int32, (_N, _NC), 1)
    nvalid = (n_row + 1) // _CBS
    msim = jnp.where(j_id < nvalid, simc, _NEG)
    mb = jnp.max(msim, axis=1, keepdims=True)
    m = jnp.maximum(mb, nl)
    eb = jnp.exp(msim - m)
    en = jnp.exp(nl - m)
    den = jnp.sum(eb, axis=1, keepdims=True) + en
    outc = (lax.dot_general(eb, cv, (((1,), (0,)), ((), ())),
                            preferred_element_type=f32)
            + en * nv_ref[0]) / den

    jm = jnp.min(jnp.where(msim == mb, j_id, _NC), axis=1, keepdims=True)
    onehot = (j_id == jm).astype(f32)
    kv4 = jnp.concatenate([k4, v4], axis=1)
    gkv = jnp.dot(onehot, kv4, preferred_element_type=f32)
    gk = gkv[:, :_CBS * _DH]
    gv = gkv[:, _CBS * _DH:]

    def seg_mat(rows, cols, group):
        r = lax.broadcasted_iota(jnp.int32, (rows, cols), 0)
        c = lax.broadcasted_iota(jnp.int32, (rows, cols), 1)
        return ((r // group if group > 1 else r % _DH) == c).astype(f32)

    s16 = seg_mat(_CBS * _DH, _CBS, _DH)
    r32 = seg_mat(_CBS * _DH, _DH, 1)
    s8 = seg_mat(_SBS * _DH, _SBS, _DH)
    r32b = seg_mat(_SBS * _DH, _DH, 1)

    qrep16 = jnp.concatenate([qf] * _CBS, axis=1)
    sim16 = jnp.dot(qrep16 * gk, s16,
                    preferred_element_type=f32) * _SCALE
    s_id = lax.broadcasted_iota(jnp.int32, (_N, _CBS), 1)
    sim16 = jnp.where(jm * _CBS + s_id <= n_row, sim16, _NEG)

    shk = [kf]
    shv = [vf]
    for t in range(1, _SBS):
        pad = jnp.zeros((t, _DH), f32)
        shk.append(jnp.concatenate([pad, kf[:_N - t]], axis=0))
        shv.append(jnp.concatenate([pad, vf[:_N - t]], axis=0))
    shk_cat = jnp.concatenate(shk, axis=1)
    shv_cat = jnp.concatenate(shv, axis=1)
    qrep8 = jnp.concatenate([qf] * _SBS, axis=1)
    d8 = jnp.dot(qrep8 * shk_cat, s8,
                 preferred_element_type=f32) * _SCALE
    t8 = lax.broadcasted_iota(jnp.int32, (_N, _SBS), 1)
    own = jnp.where(t8 <= n_row % _SBS, d8, _NEG)

    mf = jnp.maximum(jnp.max(sim16, axis=1, keepdims=True),
                     jnp.max(own, axis=1, keepdims=True))
    e16 = jnp.exp(sim16 - mf)
    e8 = jnp.exp(own - mf)
    denf = jnp.sum(e16, axis=1, keepdims=True) + jnp.sum(e8, axis=1,
                                                         keepdims=True)
    e16x = lax.dot_general(e16, s16, (((1,), (1,)), ((), ())),
                           preferred_element_type=f32)
    e8x = lax.dot_general(e8, s8, (((1,), (1,)), ((), ())),
                          preferred_element_type=f32)
    outf = (jnp.dot(e16x * gv, r32, preferred_element_type=f32)
            + jnp.dot(e8x * shv_cat, r32b,
                      preferred_element_type=f32)) / denf

    w7 = jnp.where(t8[:, :_SW] <= jnp.minimum(n_row, _SW - 1),
                   d8[:, :_SW], _NEG)
    mw = jnp.max(w7, axis=1, keepdims=True)
    ew = jnp.exp(w7 - mw)
    denw = jnp.sum(ew, axis=1, keepdims=True)
    ew8 = jnp.concatenate([ew, jnp.zeros((_N, 1), f32)], axis=1)
    ewx = lax.dot_general(ew8, s8, (((1,), (1,)), ((), ())),
                          preferred_element_type=f32)
    outw = jnp.dot(ewx * shv_cat, r32b,
                   preferred_element_type=f32) / denw

    g = g_ref[0, 0]
    comb = g[:, 0:1] * outc + g[:, 1:2] * outf + g[:, 2:3] * outw
    out_ref[0, 0] = jnp.sum(comb, axis=0, keepdims=True)


def _head_kernel(acc_ref, wo_ref, bo_ref, w1_ref, b1_ref, w2_ref, b2_ref,
                 o_ref):
    pooled = jnp.dot(acc_ref[...], wo_ref[...],
                     preferred_element_type=jnp.float32) * (1.0 / _N)
    pooled = pooled + bo_ref[...]
    h1 = jax.nn.gelu(jnp.dot(pooled, w1_ref[...],
                             preferred_element_type=jnp.float32) + b1_ref[...])
    o_ref[...] = jnp.dot(h1, w2_ref[...],
                         preferred_element_type=jnp.float32) + b2_ref[...]


@functools.partial(jax.jit, static_argnames=())
def kernel(x, W_embed, b_embed, g_norm, W_q, W_k, W_v, k_pos, v_pos, Wc_k,
           bc_k, Wc_v, bc_v, null_k, null_v, W_gate, b_gate, W_o, b_o, W_h1,
           b_h1, W_h2, b_h2):
    f32 = jnp.float32
    B, N = x.shape
    full = lambda shp: pl.BlockSpec(shp, lambda b: tuple(0 for _ in shp))

    q, k, v, gates = pl.pallas_call(
        _proj_kernel,
        grid=(B,),
        in_specs=[
            pl.BlockSpec((1, N, 1), lambda b: (b, 0, 0)),
            full((1, _DIM)), full((1, _DIM)), full((1, _DIM)),
            full((_DIM, _H * _DH)), full((_DIM, _H * _DH)),
            full((_DIM, _H * _DH)),
            full((_DIM, 3 * _H)), full((1, 3 * _H)),
        ],
        out_specs=[
            pl.BlockSpec((1, _H, N, _DH), lambda b: (b, 0, 0, 0)),
            pl.BlockSpec((1, _H, N, _DH), lambda b: (b, 0, 0, 0)),
            pl.BlockSpec((1, _H, N, _DH), lambda b: (b, 0, 0, 0)),
            pl.BlockSpec((1, N, 3 * _H), lambda b: (b, 0, 0)),
        ],
        out_shape=[
            jax.ShapeDtypeStruct((B, _H, N, _DH), f32),
            jax.ShapeDtypeStruct((B, _H, N, _DH), f32),
            jax.ShapeDtypeStruct((B, _H, N, _DH), f32),
            jax.ShapeDtypeStruct((B, N, 3 * _H), f32),
        ],
        compiler_params=pltpu.CompilerParams(
            dimension_semantics=("parallel",)),
    )(x.reshape(B, N, 1), W_embed, b_embed.reshape(1, _DIM),
      g_norm.reshape(1, _DIM), W_q, W_k, W_v, W_gate, b_gate.reshape(1, -1))

    k4 = k.reshape(B, _H, _NC, _CBS * _DH)
    v4 = v.reshape(B, _H, _NC, _CBS * _DH)
    gates_r = gates.reshape(B, N, 3, _H).transpose(0, 3, 1, 2)

    bh_spec = lambda shp: pl.BlockSpec(shp, lambda b, h: (b, h) + (0,) * (len(shp) - 2))
    h_spec = lambda shp: pl.BlockSpec(shp, lambda b, h: (h,) + (0,) * (len(shp) - 1))
    w_spec = lambda shp: pl.BlockSpec(shp, lambda b, h: tuple(0 for _ in shp))

    acc = pl.pallas_call(
        _attn_kernel,
        grid=(B, _H),
        in_specs=[
            bh_spec((1, 1, N, _DH)), bh_spec((1, 1, N, _DH)),
            bh_spec((1, 1, N, _DH)),
            bh_spec((1, 1, _NC, _CBS * _DH)), bh_spec((1, 1, _NC, _CBS * _DH)),
            h_spec((1, 1, _CBS * _DH)), h_spec((1, 1, _CBS * _DH)),
            w_spec((_CBS * _DH, _DH)), w_spec((1, _DH)),
            w_spec((_CBS * _DH, _DH)), w_spec((1, _DH)),
            h_spec((1, 1, _DH)), h_spec((1, 1, _DH)),
            bh_spec((1, 1, N, 3)),
        ],
        out_specs=pl.BlockSpec((1, 1, 1, _DH), lambda b, h: (b, h, 0, 0)),
        out_shape=jax.ShapeDtypeStruct((B, _H, 1, _DH), f32),
        compiler_params=pltpu.CompilerParams(
            dimension_semantics=("parallel", "parallel")),
    )(q, k, v, k4, v4,
      k_pos.reshape(_H, 1, _CBS * _DH), v_pos.reshape(_H, 1, _CBS * _DH),
      Wc_k, bc_k.reshape(1, _DH), Wc_v, bc_v.reshape(1, _DH),
      null_k.reshape(_H, 1, _DH), null_v.reshape(_H, 1, _DH), gates_r)

    out = pl.pallas_call(
        _head_kernel,
        out_shape=jax.ShapeDtypeStruct((B, _OUT), f32),
    )(acc.reshape(B, _H * _DH), W_o, b_o.reshape(1, _DIM),
      W_h1, b_h1.reshape(1, 32), W_h2, b_h2.reshape(1, _OUT))
    return out
